# baseline (speedup 1.0000x reference)
import functools

import jax
import jax.numpy as jnp
from jax import lax
from jax.experimental import pallas as pl
from jax.experimental.pallas import tpu as pltpu
from jax.experimental.pallas import tpu_sc as plsc

_B = 1024
_D_NUM = 256
_F = 16
_D_TOK = 32
_N_CAT = 26
_CAT_SIZE = 100000

_N_WORKERS = 32
_ROWS = _N_CAT * _D_NUM
_R_PER_W = _ROWS // _N_WORKERS
_PAD_PER_W = 256
_EMB128 = _N_CAT * _CAT_SIZE * _D_TOK // 128


def _sc_gather_body(idx_hbm, emb_hbm, out_hbm, idx_v, rows_v, sem):
    w = lax.axis_index("s") * 2 + lax.axis_index("c")
    pltpu.sync_copy(idx_hbm.at[pl.ds(2 * w, 2)], idx_v)
    cp0 = pltpu.async_copy(emb_hbm.at[idx_v.at[0]], rows_v.at[pl.ds(0, 128)], sem)
    cp1 = pltpu.async_copy(emb_hbm.at[idx_v.at[1]], rows_v.at[pl.ds(128, 128)], sem)
    cp0.wait()
    cp1.wait()
    pltpu.sync_copy(
        rows_v.at[pl.ds(0, _R_PER_W)], out_hbm.at[pl.ds(w * _R_PER_W, _R_PER_W)]
    )


@functools.cache
def _make_sc_gather():
    mesh = plsc.VectorSubcoreMesh(
        core_axis_name="c", subcore_axis_name="s", num_cores=2, num_subcores=16
    )
    return pl.kernel(
        _sc_gather_body,
        out_type=jax.ShapeDtypeStruct((_ROWS, 128), jnp.float32),
        mesh=mesh,
        scratch_types=[
            pltpu.VMEM((2, 128), jnp.int32),
            pltpu.VMEM((_PAD_PER_W, 128), jnp.float32),
            pltpu.SemaphoreType.DMA,
        ],
    )


_TILE = 128
_GRID = 9


def _tc_body(x_ref, g_ref, q_ref, w_ref, wq_ref, b_ref, out_ref):
    i = pl.program_id(0)

    @pl.when(i < _GRID - 1)
    def _dense():
        w = w_ref[...]
        out_ref[:, 0:_F, :] = jnp.broadcast_to(
            w[0:_F][None], (_TILE, _F, _D_TOK)
        )
        out_ref[:, _F:, :] = x_ref[...][:, :, None] * w[_F:][None] + b_ref[...][None]

    @pl.when(i == _GRID - 1)
    def _cat():
        out_ref[0:_N_CAT, 0:_F, :] = jnp.broadcast_to(
            wq_ref[...][None], (_N_CAT, _F, _D_TOK)
        )
        q = q_ref[...]
        e = g_ref[:, :, 0:_D_TOK]
        for c in range(1, 4):
            e = jnp.where(q == c, g_ref[:, :, c * _D_TOK:(c + 1) * _D_TOK], e)
        out_ref[0:_N_CAT, _F:, :] = e + b_ref[...][None]


def kernel(x_num, x_cat, emb, weight_, weight, bias):
    offsets = jnp.arange(_N_CAT, dtype=jnp.int32) * _CAT_SIZE
    flat = (x_cat + offsets[:, None]).reshape(-1)
    quarter = jnp.broadcast_to(
        (flat % 4).reshape(_N_CAT, _D_NUM, 1), (_N_CAT, _D_NUM, _D_TOK)
    )
    chunks = (flat // 4).reshape(_N_WORKERS, _R_PER_W)
    idx_pack = jnp.pad(chunks, ((0, 0), (0, _PAD_PER_W - _R_PER_W))).reshape(64, 128)
    emb128 = emb.reshape(_EMB128, 128)
    g = _make_sc_gather()(idx_pack, emb128)
    g3 = g.reshape(_N_CAT, _D_NUM, 128)
    return pl.pallas_call(
        _tc_body,
        grid=(_GRID,),
        in_specs=[
            pl.BlockSpec((_TILE, _D_NUM), lambda i: (jnp.minimum(i, _GRID - 2), 0)),
            pl.BlockSpec((_N_CAT, _D_NUM, 128), lambda i: (0, 0, 0)),
            pl.BlockSpec((_N_CAT, _D_NUM, _D_TOK), lambda i: (0, 0, 0)),
            pl.BlockSpec((_F + _D_NUM, _D_TOK), lambda i: (0, 0)),
            pl.BlockSpec((_F, _D_TOK), lambda i: (0, 0)),
            pl.BlockSpec((_D_NUM, _D_TOK), lambda i: (0, 0)),
        ],
        out_specs=pl.BlockSpec((_TILE, _F + _D_NUM, _D_TOK), lambda i: (i, 0, 0)),
        out_shape=jax.ShapeDtypeStruct((_B + _N_CAT, _F + _D_NUM, _D_TOK), jnp.float32),
    )(x_num, g3, quarter, weight, weight_, bias)


# speedup vs baseline: 1.4106x; 1.4106x over previous
import functools

import jax
import jax.numpy as jnp
from jax import lax
from jax.experimental import pallas as pl
from jax.experimental.pallas import tpu as pltpu
from jax.experimental.pallas import tpu_sc as plsc

_B = 1024
_D_NUM = 256
_F = 16
_D_TOK = 32
_N_CAT = 26
_CAT_SIZE = 100000

_N_WORKERS = 32
_ROWS = _N_CAT * _D_NUM
_R_PER_W = _ROWS // _N_WORKERS


_PAD_ROWS = 8192
_ROWS_PER_CORE = _PAD_ROWS // 2
_CHUNK_IR = 8
_CHUNK = _CHUNK_IR * 128


def _sc_gather_body(idx_hbm, emb_hbm, out_hbm, idx_s, sem):
    c = lax.axis_index("c")
    for chunk in range(_ROWS_PER_CORE // _CHUNK):
        irbase = c * 32 + chunk * _CHUNK_IR
        cbase = irbase * 128
        pltpu.sync_copy(idx_hbm.at[pl.ds(irbase, _CHUNK_IR)], idx_s)

        def issue(j, carry):
            r = idx_s[lax.shift_right_logical(j, 7), lax.bitwise_and(j, 127)]
            pltpu.make_async_copy(emb_hbm.at[r], out_hbm.at[cbase + j], sem).start()
            return carry

        lax.fori_loop(0, _CHUNK, issue, 0)
        pltpu.make_async_copy(
            emb_hbm.at[pl.ds(0, _CHUNK)], out_hbm.at[pl.ds(cbase, _CHUNK)], sem
        ).wait()


@functools.cache
def _make_sc_gather():
    mesh = plsc.ScalarSubcoreMesh(axis_name="c", num_cores=2)
    return pl.kernel(
        _sc_gather_body,
        out_type=jax.ShapeDtypeStruct((_PAD_ROWS, _D_TOK), jnp.float32),
        mesh=mesh,
        scratch_types=[
            pltpu.SMEM((_CHUNK_IR, 128), jnp.int32),
            pltpu.SemaphoreType.DMA,
        ],
    )


_TILE = 128
_GRID = 9


def _tc_body(x_ref, g_ref, w_ref, wq_ref, b_ref, out_ref):
    i = pl.program_id(0)

    @pl.when(i < _GRID - 1)
    def _dense():
        w = w_ref[...]
        out_ref[:, 0:_F, :] = jnp.broadcast_to(
            w[0:_F][None], (_TILE, _F, _D_TOK)
        )
        out_ref[:, _F:, :] = x_ref[...][:, :, None] * w[_F:][None] + b_ref[...][None]

    @pl.when(i == _GRID - 1)
    def _cat():
        out_ref[0:_N_CAT, 0:_F, :] = jnp.broadcast_to(
            wq_ref[...][None], (_N_CAT, _F, _D_TOK)
        )
        out_ref[0:_N_CAT, _F:, :] = g_ref[...] + b_ref[...][None]


def kernel(x_num, x_cat, emb, weight_, weight, bias):
    offsets = jnp.arange(_N_CAT, dtype=jnp.int32) * _CAT_SIZE
    flat = (x_cat + offsets[:, None]).reshape(-1)
    idx_pack = jnp.pad(flat, (0, _PAD_ROWS - _ROWS)).reshape(64, 128)
    g = _make_sc_gather()(idx_pack, emb)
    g3 = g[:_ROWS].reshape(_N_CAT, _D_NUM, _D_TOK)
    return pl.pallas_call(
        _tc_body,
        grid=(_GRID,),
        in_specs=[
            pl.BlockSpec((_TILE, _D_NUM), lambda i: (jnp.minimum(i, _GRID - 2), 0)),
            pl.BlockSpec((_N_CAT, _D_NUM, _D_TOK), lambda i: (0, 0, 0)),
            pl.BlockSpec((_F + _D_NUM, _D_TOK), lambda i: (0, 0)),
            pl.BlockSpec((_F, _D_TOK), lambda i: (0, 0)),
            pl.BlockSpec((_D_NUM, _D_TOK), lambda i: (0, 0)),
        ],
        out_specs=pl.BlockSpec((_TILE, _F + _D_NUM, _D_TOK), lambda i: (i, 0, 0)),
        out_shape=jax.ShapeDtypeStruct((_B + _N_CAT, _F + _D_NUM, _D_TOK), jnp.float32),
    )(x_num, g3, weight, weight_, bias)


# speedup vs baseline: 1.5919x; 1.1285x over previous
import functools

import jax
import jax.numpy as jnp
from jax import lax
from jax.experimental import pallas as pl
from jax.experimental.pallas import tpu as pltpu
from jax.experimental.pallas import tpu_sc as plsc

_B = 1024
_D_NUM = 256
_F = 16
_D_TOK = 32
_N_CAT = 26
_CAT_SIZE = 100000

_ROWS = _N_CAT * _D_NUM
_PAD_ROWS = 8192
_ROWS_PER_CORE = _PAD_ROWS // 2
_CHUNK_IR = 8
_CHUNK = _CHUNK_IR * 128


def _sc_gather_body(idx_hbm, emb_hbm, out_hbm, idx_s, sem):
    c = lax.axis_index("c")
    for chunk in range(_ROWS_PER_CORE // _CHUNK):
        irbase = c * 32 + chunk * _CHUNK_IR
        cbase = irbase * 128
        pltpu.sync_copy(idx_hbm.at[pl.ds(irbase, _CHUNK_IR)], idx_s)

        def issue(j, carry):
            r = idx_s[lax.shift_right_logical(j, 7), lax.bitwise_and(j, 127)]
            pltpu.make_async_copy(emb_hbm.at[r], out_hbm.at[cbase + j], sem).start()
            return carry

        lax.fori_loop(0, _CHUNK, issue, 0, unroll=8)
        pltpu.make_async_copy(
            emb_hbm.at[pl.ds(0, _CHUNK)], out_hbm.at[pl.ds(cbase, _CHUNK)], sem
        ).wait()


@functools.cache
def _make_sc_gather():
    mesh = plsc.ScalarSubcoreMesh(axis_name="c", num_cores=2)
    return pl.kernel(
        _sc_gather_body,
        out_type=jax.ShapeDtypeStruct((_PAD_ROWS, _D_TOK), jnp.float32),
        mesh=mesh,
        scratch_types=[
            pltpu.SMEM((_CHUNK_IR, 128), jnp.int32),
            pltpu.SemaphoreType.DMA,
        ],
    )


_TILE = 128
_GRID = 9


def _tc_body(xt_ref, g_ref, w_ref, wq_ref, b_ref, out_ref):
    i = pl.program_id(0)

    @pl.when(i < _GRID - 1)
    def _dense():
        xt = xt_ref[...]
        for d in range(_D_TOK):
            out_ref[0:_F, d, :] = jnp.broadcast_to(w_ref[0:_F, d : d + 1], (_F, _TILE))
            out_ref[_F:, d, :] = xt * w_ref[_F:, d : d + 1] + b_ref[:, d : d + 1]

    @pl.when(i == _GRID - 1)
    def _cat():
        for d in range(_D_TOK):
            out_ref[0:_F, d, 0:_N_CAT] = jnp.broadcast_to(
                wq_ref[:, d : d + 1], (_F, _N_CAT)
            )
            out_ref[_F:, d, 0:_N_CAT] = g_ref[:, d, :] + b_ref[:, d : d + 1]


def kernel(x_num, x_cat, emb, weight_, weight, bias):
    offsets = jnp.arange(_N_CAT, dtype=jnp.int32) * _CAT_SIZE
    flat = (x_cat + offsets[:, None]).reshape(-1)
    idx_pack = jnp.pad(flat, (0, _PAD_ROWS - _ROWS)).reshape(64, 128)
    g = _make_sc_gather()(idx_pack, emb)
    g3 = g[:_ROWS].reshape(_N_CAT, _D_NUM, _D_TOK).transpose(1, 2, 0)
    out_t = pl.pallas_call(
        _tc_body,
        grid=(_GRID,),
        in_specs=[
            pl.BlockSpec((_D_NUM, _TILE), lambda i: (0, jnp.minimum(i, _GRID - 2))),
            pl.BlockSpec((_D_NUM, _D_TOK, _N_CAT), lambda i: (0, 0, 0)),
            pl.BlockSpec((_F + _D_NUM, _D_TOK), lambda i: (0, 0)),
            pl.BlockSpec((_F, _D_TOK), lambda i: (0, 0)),
            pl.BlockSpec((_D_NUM, _D_TOK), lambda i: (0, 0)),
        ],
        out_specs=pl.BlockSpec((_F + _D_NUM, _D_TOK, _TILE), lambda i: (0, 0, i)),
        out_shape=jax.ShapeDtypeStruct((_F + _D_NUM, _D_TOK, _B + _N_CAT), jnp.float32),
    )(x_num.T, g3, weight, weight_, bias)
    return jnp.transpose(out_t, (2, 0, 1))


# speedup vs baseline: 1.6643x; 1.0455x over previous
import functools

import jax
import jax.numpy as jnp
from jax import lax
from jax.experimental import pallas as pl
from jax.experimental.pallas import tpu as pltpu
from jax.experimental.pallas import tpu_sc as plsc

_B = 1024
_D_NUM = 256
_F = 16
_D_TOK = 32
_N_CAT = 26
_CAT_SIZE = 100000

_ROWS = _N_CAT * _D_NUM
_PAD_ROWS = 8192
_ROWS_PER_CORE = _PAD_ROWS // 2
_CHUNK_IR = 8
_CHUNK = _CHUNK_IR * 128


_R_PER_W = _ROWS // 32


def _sc_gather_body(idx_hbm, emb_hbm, out_hbm, idx_v, sem):
    w = lax.axis_index("s") * 2 + lax.axis_index("c")
    base = w * _R_PER_W
    pltpu.sync_copy(idx_hbm.at[pl.ds(2 * w, 2)], idx_v)
    lanes = lax.iota(jnp.int32, 16)
    zeros = jnp.zeros((16,), jnp.int32)
    for j in range(_R_PER_W):
        h, l = divmod(j, 128)
        v16 = idx_v[h, pl.ds((l // 16) * 16, 16)]
        r = lax.reduce_sum_p.bind(
            jnp.where(lanes == l % 16, v16, zeros), axes=(0,)
        )
        pltpu.make_async_copy(emb_hbm.at[r], out_hbm.at[base + j], sem).start()
    pltpu.make_async_copy(
        emb_hbm.at[pl.ds(0, _R_PER_W)], out_hbm.at[pl.ds(base, _R_PER_W)], sem
    ).wait()


@functools.cache
def _make_sc_gather():
    mesh = plsc.VectorSubcoreMesh(
        core_axis_name="c", subcore_axis_name="s", num_cores=2, num_subcores=16
    )
    return pl.kernel(
        _sc_gather_body,
        out_type=jax.ShapeDtypeStruct((_ROWS, _D_TOK), jnp.float32),
        mesh=mesh,
        scratch_types=[
            pltpu.VMEM((2, 128), jnp.int32),
            pltpu.SemaphoreType.DMA,
        ],
        compiler_params=pltpu.CompilerParams(needs_layout_passes=False),
    )


_TILE = 128
_GRID = 9


def _tc_body(xt_ref, g_ref, w_ref, wq_ref, b_ref, out_ref):
    i = pl.program_id(0)

    @pl.when(i < _GRID - 1)
    def _dense():
        xt = xt_ref[...]
        for d in range(_D_TOK):
            out_ref[0:_F, d, :] = jnp.broadcast_to(w_ref[0:_F, d : d + 1], (_F, _TILE))
            out_ref[_F:, d, :] = xt * w_ref[_F:, d : d + 1] + b_ref[:, d : d + 1]

    @pl.when(i == _GRID - 1)
    def _cat():
        for d in range(_D_TOK):
            out_ref[0:_F, d, 0:_N_CAT] = jnp.broadcast_to(
                wq_ref[:, d : d + 1], (_F, _N_CAT)
            )
            out_ref[_F:, d, 0:_N_CAT] = g_ref[:, d, :] + b_ref[:, d : d + 1]


def kernel(x_num, x_cat, emb, weight_, weight, bias):
    offsets = jnp.arange(_N_CAT, dtype=jnp.int32) * _CAT_SIZE
    flat = (x_cat + offsets[:, None]).reshape(-1)
    chunks = flat.reshape(32, _R_PER_W)
    idx_pack = jnp.pad(chunks, ((0, 0), (0, 256 - _R_PER_W))).reshape(64, 128)
    g = _make_sc_gather()(idx_pack, emb)
    g3 = g.reshape(_N_CAT, _D_NUM, _D_TOK).transpose(1, 2, 0)
    out_t = pl.pallas_call(
        _tc_body,
        grid=(_GRID,),
        in_specs=[
            pl.BlockSpec((_D_NUM, _TILE), lambda i: (0, jnp.minimum(i, _GRID - 2))),
            pl.BlockSpec((_D_NUM, _D_TOK, _N_CAT), lambda i: (0, 0, 0)),
            pl.BlockSpec((_F + _D_NUM, _D_TOK), lambda i: (0, 0)),
            pl.BlockSpec((_F, _D_TOK), lambda i: (0, 0)),
            pl.BlockSpec((_D_NUM, _D_TOK), lambda i: (0, 0)),
        ],
        out_specs=pl.BlockSpec((_F + _D_NUM, _D_TOK, _TILE), lambda i: (0, 0, i)),
        out_shape=jax.ShapeDtypeStruct((_F + _D_NUM, _D_TOK, _B + _N_CAT), jnp.float32),
    )(x_num.T, g3, weight, weight_, bias)
    return jnp.transpose(out_t, (2, 0, 1))


# speedup vs baseline: 1.7042x; 1.0240x over previous
import functools

import jax
import jax.numpy as jnp
from jax import lax
from jax.experimental import pallas as pl
from jax.experimental.pallas import tpu as pltpu
from jax.experimental.pallas import tpu_sc as plsc

_B = 1024
_D_NUM = 256
_F = 16
_D_TOK = 32
_N_CAT = 26
_CAT_SIZE = 100000

_ROWS = _N_CAT * _D_NUM
_PAD_ROWS = 8192
_ROWS_PER_CORE = _PAD_ROWS // 2
_CHUNK_IR = 8
_CHUNK = _CHUNK_IR * 128


_R_PER_W = _ROWS // 32


def _sc_gather_body(idx_hbm, emb_hbm, out_hbm, idx_v, sem):
    w = lax.axis_index("s") * 2 + lax.axis_index("c")
    base = w * _R_PER_W
    pltpu.sync_copy(idx_hbm.at[pl.ds(2 * w, 2)], idx_v)
    lanes = lax.iota(jnp.int32, 16)
    zeros = jnp.zeros((16,), jnp.int32)
    for j in range(_R_PER_W):
        h, l = divmod(j, 128)
        v16 = idx_v[h, pl.ds((l // 16) * 16, 16)]
        r = lax.reduce_sum_p.bind(
            jnp.where(lanes == l % 16, v16, zeros), axes=(0,)
        )
        pltpu.make_async_copy(emb_hbm.at[r], out_hbm.at[base + j], sem).start()
    pltpu.make_async_copy(
        emb_hbm.at[pl.ds(0, _R_PER_W)], out_hbm.at[pl.ds(base, _R_PER_W)], sem
    ).wait()


@functools.cache
def _make_sc_gather():
    mesh = plsc.VectorSubcoreMesh(
        core_axis_name="c", subcore_axis_name="s", num_cores=2, num_subcores=16
    )
    return pl.kernel(
        _sc_gather_body,
        out_type=jax.ShapeDtypeStruct((_ROWS, _D_TOK), jnp.float32),
        mesh=mesh,
        scratch_types=[
            pltpu.VMEM((2, 128), jnp.int32),
            pltpu.SemaphoreType.DMA,
        ],
        compiler_params=pltpu.CompilerParams(needs_layout_passes=False),
    )


_TILE = 128
_GRID = 9


def _tc_body(xt_ref, g_ref, wt_ref, wqt_ref, bt_ref, out_ref):
    i = pl.program_id(0)

    def bdim(src, shape, dims):
        return lax.broadcast_in_dim(src, shape, dims)

    @pl.when(i < _GRID - 1)
    def _dense():
        out_ref[0:_F, :, :] = bdim(wt_ref[0:_F], (_F, _D_TOK, _TILE), (0, 1))
        out_ref[_F:, :, :] = (
            bdim(xt_ref[...], (_D_NUM, _D_TOK, _TILE), (0, 2))
            * bdim(wt_ref[_F:], (_D_NUM, _D_TOK, _TILE), (0, 1))
            + bdim(bt_ref[...], (_D_NUM, _D_TOK, _TILE), (0, 1))
        )

    @pl.when(i == _GRID - 1)
    def _cat():
        out_ref[0:_F, :, 0:_N_CAT] = bdim(wqt_ref[...], (_F, _D_TOK, _N_CAT), (0, 1))
        out_ref[_F:, :, 0:_N_CAT] = g_ref[...] + bdim(
            bt_ref[...], (_D_NUM, _D_TOK, _N_CAT), (0, 1)
        )


def kernel(x_num, x_cat, emb, weight_, weight, bias):
    offsets = jnp.arange(_N_CAT, dtype=jnp.int32) * _CAT_SIZE
    flat = (x_cat + offsets[:, None]).reshape(-1)
    chunks = flat.reshape(32, _R_PER_W)
    idx_pack = jnp.pad(chunks, ((0, 0), (0, 256 - _R_PER_W))).reshape(64, 128)
    g = _make_sc_gather()(idx_pack, emb)
    g3 = g.reshape(_N_CAT, _D_NUM, _D_TOK).transpose(1, 2, 0)
    out_t = pl.pallas_call(
        _tc_body,
        grid=(_GRID,),
        in_specs=[
            pl.BlockSpec((_D_NUM, _TILE), lambda i: (0, jnp.minimum(i, _GRID - 2))),
            pl.BlockSpec((_D_NUM, _D_TOK, _N_CAT), lambda i: (0, 0, 0)),
            pl.BlockSpec((_F + _D_NUM, _D_TOK), lambda i: (0, 0)),
            pl.BlockSpec((_F, _D_TOK), lambda i: (0, 0)),
            pl.BlockSpec((_D_NUM, _D_TOK), lambda i: (0, 0)),
        ],
        out_specs=pl.BlockSpec((_F + _D_NUM, _D_TOK, _TILE), lambda i: (0, 0, i)),
        out_shape=jax.ShapeDtypeStruct((_F + _D_NUM, _D_TOK, _B + _N_CAT), jnp.float32),
    )(x_num.T, g3, weight, weight_, bias)
    return jnp.transpose(out_t, (2, 0, 1))
